# Initial kernel scaffold; baseline (speedup 1.0000x reference)
import jax
import jax.numpy as jnp
from jax.experimental import pallas as pl


def kernel(nodes, features, adj_r1, adj_r2, adj_r3, weight, a):
    raise NotImplementedError("write your pallas kernel here")


# speedup vs baseline: 1.3624x; 1.3624x over previous
import functools

import jax
import jax.numpy as jnp
from jax import lax
from jax.experimental import pallas as pl
from jax.experimental.pallas import tpu as pltpu
from jax.experimental.pallas import tpu_sc as plsc

N_NODES = 100000
B = 8192
DEG = 16
F = 190
FP = 256
E = 64
K = 8

NC = 2
NS = 16
NW = NC * NS
PER_W = B // NW
CHUNK = 8
NCHUNK = PER_W // CHUNK
NFC = 12


@functools.lru_cache(maxsize=None)
def _make_sc_stage():
    mesh = plsc.VectorSubcoreMesh(core_axis_name="c", subcore_axis_name="s")
    return pl.kernel(
        _sc_stage_body,
        mesh=mesh,
        compiler_params=pltpu.CompilerParams(needs_layout_passes=False),
        out_type=[
            jax.ShapeDtypeStruct((B, FP), jnp.float32),
            jax.ShapeDtypeStruct((B, FP), jnp.float32),
            jax.ShapeDtypeStruct((B, FP), jnp.float32),
            jax.ShapeDtypeStruct((B, FP), jnp.float32),
        ],
        scratch_types=[
            pltpu.VMEM((PER_W,), jnp.int32),
            pltpu.VMEM((PER_W * DEG,), jnp.int32),
            pltpu.VMEM((PER_W * DEG,), jnp.int32),
            pltpu.VMEM((PER_W * DEG,), jnp.int32),
            pltpu.VMEM((CHUNK * DEG, FP), jnp.float32),
            pltpu.VMEM((CHUNK * DEG, FP), jnp.float32),
            pltpu.VMEM((CHUNK * DEG, FP), jnp.float32),
            pltpu.VMEM((CHUNK, FP), jnp.float32),
            pltpu.VMEM((CHUNK, FP), jnp.float32),
            pltpu.VMEM((CHUNK, FP), jnp.float32),
            pltpu.VMEM((CHUNK, FP), jnp.float32),
            pltpu.SemaphoreType.DMA,
            pltpu.SemaphoreType.DMA,
            pltpu.SemaphoreType.DMA,
            pltpu.SemaphoreType.DMA,
        ],
    )


def _sc_stage_body(nodes_h, adj1_h, adj2_h, adj3_h, feat_h,
                   r1_h, r2_h, r3_h, cent_h,
                   nodes_v, adj1_v, adj2_v, adj3_v,
                   rows1_v, rows2_v, rows3_v, crows_v,
                   rf1_v, rf2_v, rf3_v,
                   sem0, sem1, sem2, sem3):
    wid = lax.axis_index("s") * NC + lax.axis_index("c")
    base = wid * PER_W

    pltpu.sync_copy(nodes_h.at[pl.ds(base, PER_W)], nodes_v)
    pltpu.sync_copy(adj1_h.at[pl.ds(base * DEG, PER_W * DEG)], adj1_v)
    pltpu.sync_copy(adj2_h.at[pl.ds(base * DEG, PER_W * DEG)], adj2_v)
    pltpu.sync_copy(adj3_h.at[pl.ds(base * DEG, PER_W * DEG)], adj3_v)

    d_iota = lax.iota(jnp.int32, 16)
    zero16 = jnp.zeros((16,), jnp.float32)
    one16 = jnp.ones((16,), jnp.float32)

    def bf16r(x):
        xi = plsc.bitcast(x, jnp.int32)
        xi = (xi + 0x7FFF + ((xi >> 16) & 1)) & jnp.int32(-65536)
        return plsc.bitcast(xi, jnp.float32)

    def chunk_body(c, carry):
        hc = pltpu.async_copy(
            feat_h.at[nodes_v.at[pl.ds(c * CHUNK, CHUNK)]], crows_v, sem0)
        h1 = pltpu.async_copy(
            feat_h.at[adj1_v.at[pl.ds(c * CHUNK * DEG, CHUNK * DEG)]],
            rows1_v, sem1)
        h2 = pltpu.async_copy(
            feat_h.at[adj2_v.at[pl.ds(c * CHUNK * DEG, CHUNK * DEG)]],
            rows2_v, sem2)
        h3 = pltpu.async_copy(
            feat_h.at[adj3_v.at[pl.ds(c * CHUNK * DEG, CHUNK * DEG)]],
            rows3_v, sem3)
        hc.wait()
        h1.wait()
        h2.wait()
        h3.wait()

        def center_body(i, carry2):
            cvecs = [bf16r(crows_v[i, pl.ds(k * 16, 16)])
                     for k in range(NFC)]

            for rows_v, rf_v in ((rows1_v, rf1_v), (rows2_v, rf2_v),
                                 (rows3_v, rf3_v)):
                def row_body(d, carry3, rows_v=rows_v, cvecs=cvecs):
                    dotv, nnv = carry3
                    rd = i * DEG + d
                    acc_d = zero16
                    acc_n = zero16
                    for k in range(NFC):
                        v = rows_v[rd, pl.ds(k * 16, 16)]
                        acc_d = acc_d + bf16r(v) * cvecs[k]
                        acc_n = acc_n + v * v
                    lane = d_iota == d
                    dotv = jnp.where(lane, jnp.sum(acc_d), dotv)
                    nnv = jnp.where(lane, jnp.sum(acc_n), nnv)
                    return (dotv, nnv)

                dotv, nnv = lax.fori_loop(
                    0, DEG, row_body, (zero16, one16))
                sim = dotv * jnp.abs(dotv) / nnv
                _, sv = plsc.sort_key_val(sim, d_iota, descending=True)

                accs = [zero16] * NFC
                for j in range(K):
                    rj = i * DEG + sv[j]
                    for k in range(NFC):
                        accs[k] = accs[k] + rows_v[rj, pl.ds(k * 16, 16)]
                for k in range(NFC):
                    rf_v[i, pl.ds(k * 16, 16)] = jnp.maximum(
                        accs[k] * (1.0 / K), 0.0)
            return carry2

        lax.fori_loop(0, CHUNK, center_body, 0)

        gb = base + c * CHUNK
        pltpu.sync_copy(crows_v, cent_h.at[pl.ds(gb, CHUNK)])
        pltpu.sync_copy(rf1_v, r1_h.at[pl.ds(gb, CHUNK)])
        pltpu.sync_copy(rf2_v, r2_h.at[pl.ds(gb, CHUNK)])
        pltpu.sync_copy(rf3_v, r3_h.at[pl.ds(gb, CHUNK)])
        return carry

    lax.fori_loop(0, NCHUNK, chunk_body, 0)


BLK = 1024
GRID = B // BLK


def _tc_body(cent_ref, r1_ref, r2_ref, r3_ref, w_ref, a_ref,
             out_ref, att_ref, acc_ref):
    pid = pl.program_id(0)

    @pl.when(pid == 0)
    def _():
        acc_ref[0] = 0.0
        acc_ref[1] = 0.0
        acc_ref[2] = 0.0

    bf = jnp.bfloat16
    w = w_ref[...].astype(bf)
    ch = jnp.dot(cent_ref[...].astype(bf), w,
                 preferred_element_type=jnp.float32)
    nh1 = jnp.dot(r1_ref[...].astype(bf), w,
                  preferred_element_type=jnp.float32)
    nh2 = jnp.dot(r2_ref[...].astype(bf), w,
                  preferred_element_type=jnp.float32)
    nh3 = jnp.dot(r3_ref[...].astype(bf), w,
                  preferred_element_type=jnp.float32)
    a1 = a_ref[0:E, :].astype(bf)
    a2 = a_ref[E:2 * E, :].astype(bf)
    eb = jnp.dot(ch.astype(bf), a1, preferred_element_type=jnp.float32)

    def leaky(x):
        return jnp.where(x >= 0, x, 0.2 * x)

    e1 = leaky(eb + jnp.dot(nh1.astype(bf), a2,
                            preferred_element_type=jnp.float32))
    e2 = leaky(eb + jnp.dot(nh2.astype(bf), a2,
                            preferred_element_type=jnp.float32))
    e3 = leaky(eb + jnp.dot(nh3.astype(bf), a2,
                            preferred_element_type=jnp.float32))
    m = jnp.maximum(jnp.maximum(e1, e2), e3)
    x1 = jnp.exp(e1 - m)
    x2 = jnp.exp(e2 - m)
    x3 = jnp.exp(e3 - m)
    s = x1 + x2 + x3
    at1 = x1 / s
    at2 = x2 / s
    at3 = x3 / s
    out_ref[...] = jnp.maximum(ch + at1 * nh1 + at2 * nh2 + at3 * nh3, 0.0)

    acc_ref[0] += jnp.sum(at1)
    acc_ref[1] += jnp.sum(at2)
    acc_ref[2] += jnp.sum(at3)

    @pl.when(pid == GRID - 1)
    def _():
        s0 = acc_ref[0]
        s1 = acc_ref[1]
        s2 = acc_ref[2]
        mm = jnp.maximum(s0, jnp.maximum(s1, s2))
        z0 = jnp.exp(s0 - mm)
        z1 = jnp.exp(s1 - mm)
        z2 = jnp.exp(s2 - mm)
        zs = z0 + z1 + z2
        li = lax.broadcasted_iota(jnp.int32, (1, 128), 1)
        att_ref[...] = jnp.where(
            li == 0, z0 / zs,
            jnp.where(li == 1, z1 / zs, jnp.where(li == 2, z2 / zs, 0.0)))


def _tc_stage(cents, r1, r2, r3, weight, a):
    return pl.pallas_call(
        _tc_body,
        grid=(GRID,),
        in_specs=[
            pl.BlockSpec((BLK, FP), lambda i: (i, 0)),
            pl.BlockSpec((BLK, FP), lambda i: (i, 0)),
            pl.BlockSpec((BLK, FP), lambda i: (i, 0)),
            pl.BlockSpec((BLK, FP), lambda i: (i, 0)),
            pl.BlockSpec((FP, E), lambda i: (0, 0)),
            pl.BlockSpec((2 * E, 1), lambda i: (0, 0)),
        ],
        out_specs=[
            pl.BlockSpec((BLK, E), lambda i: (i, 0)),
            pl.BlockSpec((1, 128), lambda i: (0, 0)),
        ],
        out_shape=[
            jax.ShapeDtypeStruct((B, E), jnp.float32),
            jax.ShapeDtypeStruct((1, 128), jnp.float32),
        ],
        scratch_shapes=[pltpu.SMEM((3,), jnp.float32)],
    )(cents, r1, r2, r3, weight, a)


def kernel(nodes, features, adj_r1, adj_r2, adj_r3, weight, a):
    fpad = jnp.pad(features, ((0, 0), (0, FP - F)))
    wpad = jnp.pad(weight, ((0, FP - F), (0, 0)))
    r1, r2, r3, cents = _make_sc_stage()(
        nodes, adj_r1.reshape(-1), adj_r2.reshape(-1), adj_r3.reshape(-1),
        fpad)
    combined, att_row = _tc_stage(cents, r1, r2, r3, wpad, a)
    return (combined, r1[:, :F], r2[:, :F], r3[:, :F], att_row[0, :3])
